# baseline (speedup 1.0000x reference)
import functools

import jax
import jax.numpy as jnp
from jax import lax
from jax.experimental import pallas as pl
from jax.experimental.pallas import tpu as pltpu
from jax.experimental.pallas import tpu_sc as plsc

_VOCAB = 1000000
_DIM = 64
_BATCH = 4096
_HIST = 200

_NC = 2
_NS = 16
_NW = _NC * _NS

_B_TOTAL = _BATCH * _HIST
_B_PER_W = _B_TOTAL // _NW
_CHUNK = 128
_N_CHUNKS = _B_PER_W // _CHUNK


@functools.partial(
    pl.kernel,
    out_type=jax.ShapeDtypeStruct((_B_TOTAL, 2, _DIM), jnp.float32),
    mesh=plsc.VectorSubcoreMesh(
        core_axis_name="c", subcore_axis_name="s", num_cores=_NC, num_subcores=_NS
    ),
    compiler_params=pltpu.CompilerParams(use_tc_tiling_on_sc=False),
    scratch_types=[
        pltpu.VMEM((_N_CHUNKS, _CHUNK), jnp.int32),
        pltpu.VMEM((2, _CHUNK, _DIM), jnp.float32),
        pltpu.SemaphoreType.DMA,
        pltpu.SemaphoreType.DMA,
    ],
)
def _gather_kernel(table_hbm, tok_hbm, out_hbm, idx_v, rows_v, gsem, wsem):
    wid = lax.axis_index("s") * _NC + lax.axis_index("c")
    base = wid * _B_PER_W

    pltpu.sync_copy(tok_hbm.at[wid], idx_v)

    pltpu.async_copy(table_hbm.at[idx_v.at[0]], rows_v.at[0], gsem)

    def chunk_body(i, _):
        slot = lax.rem(i, 2)
        nxt = lax.rem(i + 1, 2)

        @pl.when(i >= 1)
        def _():
            pltpu.make_async_copy(
                rows_v.at[nxt],
                out_hbm.at[pl.ds(base + (i - 1) * _CHUNK, _CHUNK), 0],
                wsem,
            ).wait()

        @pl.when(i + 1 < _N_CHUNKS)
        def _():
            pltpu.async_copy(
                table_hbm.at[idx_v.at[i + 1]], rows_v.at[nxt], gsem
            )

        pltpu.make_async_copy(
            table_hbm.at[idx_v.at[i]], rows_v.at[slot], gsem
        ).wait()

        pltpu.async_copy(
            rows_v.at[slot],
            out_hbm.at[pl.ds(base + i * _CHUNK, _CHUNK), 0],
            wsem,
        )
        return 0

    lax.fori_loop(0, _N_CHUNKS, chunk_body, 0)

    pltpu.make_async_copy(
        rows_v.at[(_N_CHUNKS - 1) % 2],
        out_hbm.at[pl.ds(base + (_N_CHUNKS - 1) * _CHUNK, _CHUNK), 0],
        wsem,
    ).wait()


def kernel(tokens, embedding_weight):
    tab = jnp.pad(embedding_weight, ((0, 0), (0, _DIM))).reshape(2 * _VOCAB, _DIM)
    tok = (tokens.astype(jnp.int32) * 2).reshape(_NW, _N_CHUNKS, _CHUNK)
    out = _gather_kernel(tab, tok)
    return out[:, 0, :].reshape(_BATCH, _HIST, _DIM)


# speedup vs baseline: 1.7833x; 1.7833x over previous
import functools

import jax
import jax.numpy as jnp
from jax import lax
from jax.experimental import pallas as pl
from jax.experimental.pallas import tpu as pltpu
from jax.experimental.pallas import tpu_sc as plsc

_VOCAB = 1000000
_DIM = 64
_BATCH = 4096
_HIST = 200

_NC = 2
_NS = 16
_NW = _NC * _NS
_BB = _BATCH // _NW


@functools.partial(
    pl.kernel,
    out_type=jax.ShapeDtypeStruct((_HIST, 8, _NW, 8, _BB), jnp.float32),
    mesh=plsc.VectorSubcoreMesh(
        core_axis_name="c", subcore_axis_name="s", num_cores=_NC, num_subcores=_NS
    ),
    compiler_params=pltpu.CompilerParams(use_tc_tiling_on_sc=False, needs_layout_passes=False),
    scratch_types=[
        pltpu.VMEM((_HIST, _BB), jnp.int32),
        pltpu.VMEM((2, _BB, _DIM), jnp.float32),
        pltpu.VMEM((2, 8, 8, _BB), jnp.float32),
        pltpu.SemaphoreType.DMA,
        pltpu.SemaphoreType.DMA,
    ],
)
def _gather_t_kernel(table_hbm, tok_hbm, out_hbm, idx_v, rows_v, trows_v, gsem, wsem):
    wid = lax.axis_index("s") * _NC + lax.axis_index("c")

    pltpu.sync_copy(tok_hbm.at[wid], idx_v)

    pltpu.async_copy(table_hbm.at[idx_v.at[0]], rows_v.at[0], gsem)

    lane = lax.iota(jnp.int32, 16)

    def t_body(t, _):
        slot = lax.rem(t, 2)
        nxt = lax.rem(t + 1, 2)

        @pl.when(t >= 2)
        def _():
            pltpu.make_async_copy(
                trows_v.at[slot], out_hbm.at[t - 2, :, wid], wsem
            ).wait()

        @pl.when(t + 1 < _HIST)
        def _():
            pltpu.async_copy(table_hbm.at[idx_v.at[t + 1]], rows_v.at[nxt], gsem)

        pltpu.make_async_copy(table_hbm.at[idx_v.at[t]], rows_v.at[slot], gsem).wait()

        rows = rows_v.at[slot]

        def f_body(f, _):
            fb = f // 8
            fi = lax.rem(f, 8)
            col = jnp.broadcast_to(f, (16,))
            for j in range(8):
                vals = plsc.load_gather(rows, [lane + (16 * j), col])
                trows_v[slot, fb, fi, pl.ds(16 * j, 16)] = vals
            return 0

        lax.fori_loop(0, _DIM, f_body, 0)

        pltpu.async_copy(trows_v.at[slot], out_hbm.at[t, :, wid], wsem)
        return 0

    lax.fori_loop(0, _HIST, t_body, 0)

    pltpu.make_async_copy(
        trows_v.at[(_HIST - 2) % 2], out_hbm.at[_HIST - 2, :, wid], wsem
    ).wait()
    pltpu.make_async_copy(
        trows_v.at[(_HIST - 1) % 2], out_hbm.at[_HIST - 1, :, wid], wsem
    ).wait()


def kernel(tokens, embedding_weight):
    tab = jnp.pad(embedding_weight, ((0, 0), (0, _DIM))).reshape(2 * _VOCAB, _DIM)
    tok = (tokens.astype(jnp.int32) * 2).reshape(_NW, _BB, _HIST).transpose(0, 2, 1)
    out5 = _gather_t_kernel(tab, tok)
    return (
        out5.transpose(2, 4, 0, 1, 3).reshape(_BATCH, _HIST, _DIM)
    )


# speedup vs baseline: 2.6236x; 1.4712x over previous
import functools

import jax
import jax.numpy as jnp
from jax import lax
from jax.experimental import pallas as pl
from jax.experimental.pallas import tpu as pltpu
from jax.experimental.pallas import tpu_sc as plsc

_VOCAB = 1000000
_DIM = 64
_BATCH = 4096
_HIST = 200

_NC = 2
_NS = 16
_NW = _NC * _NS

_B_TOTAL = _BATCH * _HIST
_B_PER_W = _B_TOTAL // _NW
_CHUNK = 128
_N_CHUNKS = _B_PER_W // _CHUNK


@functools.partial(
    pl.kernel,
    out_type=jax.ShapeDtypeStruct((_B_TOTAL, _DIM), jnp.float32),
    mesh=plsc.VectorSubcoreMesh(
        core_axis_name="c", subcore_axis_name="s", num_cores=_NC, num_subcores=_NS
    ),
    compiler_params=pltpu.CompilerParams(use_tc_tiling_on_sc=False),
    scratch_types=[
        pltpu.VMEM((_N_CHUNKS, _CHUNK), jnp.int32),
        pltpu.VMEM((2, _CHUNK, _DIM), jnp.float32),
        pltpu.SemaphoreType.DMA,
        pltpu.SemaphoreType.DMA,
    ],
)
def _gather_kernel(table_hbm, tok_hbm, out_hbm, idx_v, rows_v, gsem, wsem):
    wid = lax.axis_index("s") * _NC + lax.axis_index("c")
    base = wid * _B_PER_W

    pltpu.sync_copy(tok_hbm.at[wid], idx_v)

    pltpu.async_copy(table_hbm.at[idx_v.at[0]], rows_v.at[0], gsem)

    def chunk_body(i, _):
        slot = lax.rem(i, 2)
        nxt = lax.rem(i + 1, 2)

        @pl.when(i >= 1)
        def _():
            pltpu.make_async_copy(
                rows_v.at[nxt],
                out_hbm.at[pl.ds(base + (i - 1) * _CHUNK, _CHUNK)],
                wsem,
            ).wait()

        @pl.when(i + 1 < _N_CHUNKS)
        def _():
            pltpu.async_copy(
                table_hbm.at[idx_v.at[i + 1]], rows_v.at[nxt], gsem
            )

        pltpu.make_async_copy(
            table_hbm.at[idx_v.at[i]], rows_v.at[slot], gsem
        ).wait()

        pltpu.async_copy(
            rows_v.at[slot],
            out_hbm.at[pl.ds(base + i * _CHUNK, _CHUNK)],
            wsem,
        )
        return 0

    lax.fori_loop(0, _N_CHUNKS, chunk_body, 0)

    pltpu.make_async_copy(
        rows_v.at[(_N_CHUNKS - 1) % 2],
        out_hbm.at[pl.ds(base + (_N_CHUNKS - 1) * _CHUNK, _CHUNK)],
        wsem,
    ).wait()


def kernel(tokens, embedding_weight):
    tab = jnp.pad(embedding_weight, ((0, 0), (0, _DIM))).reshape(2 * _VOCAB, _DIM)
    tok = (tokens.astype(jnp.int32) * 2).reshape(_NW, _N_CHUNKS, _CHUNK)
    out = _gather_kernel(tab, tok)
    return out.reshape(_BATCH, _HIST, _DIM)


# speedup vs baseline: 4.8091x; 1.8330x over previous
import functools

import jax
import jax.numpy as jnp
from jax import lax
from jax.experimental import pallas as pl
from jax.experimental.pallas import tpu as pltpu
from jax.experimental.pallas import tpu_sc as plsc

_VOCAB = 1000000
_DIM = 64
_BATCH = 4096
_HIST = 200

_NC = 2
_NS = 16
_NW = _NC * _NS

_B_TOTAL = _BATCH * _HIST
_B_PER_W = _B_TOTAL // _NW
_CHUNK = 128
_N_CHUNKS = _B_PER_W // _CHUNK

_FC = 6400
_FGRID = -(-_VOCAB // _FC)


def _fmt_body(tT_ref, out_ref):
    x = tT_ref[...]
    out_ref[:, 0:_DIM] = x.T


_format = pl.pallas_call(
    _fmt_body,
    grid=(_FGRID,),
    in_specs=[pl.BlockSpec((_DIM, _FC), lambda i: (0, i))],
    out_specs=pl.BlockSpec((_FC, 128), lambda i: (i, 0)),
    out_shape=jax.ShapeDtypeStruct((_VOCAB, 128), jnp.float32),
)


@functools.partial(
    pl.kernel,
    out_type=jax.ShapeDtypeStruct((_B_TOTAL, 2 * _DIM), jnp.float32),
    mesh=plsc.VectorSubcoreMesh(
        core_axis_name="c", subcore_axis_name="s", num_cores=_NC, num_subcores=_NS
    ),
    compiler_params=pltpu.CompilerParams(use_tc_tiling_on_sc=False),
    scratch_types=[
        pltpu.VMEM((_N_CHUNKS, _CHUNK), jnp.int32),
        pltpu.VMEM((2, _CHUNK, _DIM), jnp.float32),
        pltpu.SemaphoreType.DMA,
        pltpu.SemaphoreType.DMA,
    ],
)
def _gather_kernel(table_hbm, tok_hbm, out_hbm, idx_v, rows_v, gsem, wsem):
    wid = lax.axis_index("s") * _NC + lax.axis_index("c")
    base = wid * _B_PER_W

    pltpu.sync_copy(tok_hbm.at[wid], idx_v)

    pltpu.async_copy(table_hbm.at[idx_v.at[0]], rows_v.at[0], gsem)

    def chunk_body(i, _):
        slot = lax.rem(i, 2)
        nxt = lax.rem(i + 1, 2)

        @pl.when(i >= 1)
        def _():
            pltpu.make_async_copy(
                rows_v.at[nxt],
                out_hbm.at[pl.ds(base + (i - 1) * _CHUNK, _CHUNK), pl.ds(0, _DIM)],
                wsem,
            ).wait()

        @pl.when(i + 1 < _N_CHUNKS)
        def _():
            pltpu.async_copy(
                table_hbm.at[idx_v.at[i + 1]], rows_v.at[nxt], gsem
            )

        pltpu.make_async_copy(
            table_hbm.at[idx_v.at[i]], rows_v.at[slot], gsem
        ).wait()

        pltpu.async_copy(
            rows_v.at[slot],
            out_hbm.at[pl.ds(base + i * _CHUNK, _CHUNK), pl.ds(0, _DIM)],
            wsem,
        )
        return 0

    lax.fori_loop(0, _N_CHUNKS, chunk_body, 0)

    pltpu.make_async_copy(
        rows_v.at[(_N_CHUNKS - 1) % 2],
        out_hbm.at[pl.ds(base + (_N_CHUNKS - 1) * _CHUNK, _CHUNK), pl.ds(0, _DIM)],
        wsem,
    ).wait()


def kernel(tokens, embedding_weight):
    tab_fmt = _format(embedding_weight.T)
    tab = tab_fmt.reshape(2 * _VOCAB, _DIM)
    tok = (tokens.astype(jnp.int32) * 2).reshape(_NW, _N_CHUNKS, _CHUNK)
    out = _gather_kernel(tab, tok)
    return out.reshape(_BATCH, _HIST, 2 * _DIM)[:, :, :_DIM]


# speedup vs baseline: 5.0808x; 1.0565x over previous
import functools

import jax
import jax.numpy as jnp
from jax import lax
from jax.experimental import pallas as pl
from jax.experimental.pallas import tpu as pltpu
from jax.experimental.pallas import tpu_sc as plsc

_VOCAB = 1000000
_DIM = 64
_BATCH = 4096
_HIST = 200

_NC = 2
_NS = 16
_NW = _NC * _NS

_B_TOTAL = _BATCH * _HIST
_B_PER_W = _B_TOTAL // _NW
_CHUNK = 128
_N_CHUNKS = _B_PER_W // _CHUNK

_FC = 12800
_FGRID = -(-_VOCAB // _FC)


def _fmt_body(tT_ref, out_ref):
    x = tT_ref[...]
    out_ref[:, 0:_DIM] = x.T


_format = pl.pallas_call(
    _fmt_body,
    grid=(_FGRID,),
    in_specs=[pl.BlockSpec((_DIM, _FC), lambda i: (0, i))],
    out_specs=pl.BlockSpec((_FC, 128), lambda i: (i, 0)),
    out_shape=jax.ShapeDtypeStruct((_VOCAB, 128), jnp.float32),
)


@functools.partial(
    pl.kernel,
    out_type=jax.ShapeDtypeStruct((_B_TOTAL, 2 * _DIM), jnp.float32),
    mesh=plsc.VectorSubcoreMesh(
        core_axis_name="c", subcore_axis_name="s", num_cores=_NC, num_subcores=_NS
    ),
    compiler_params=pltpu.CompilerParams(use_tc_tiling_on_sc=False),
    scratch_types=[
        pltpu.VMEM((_N_CHUNKS, _CHUNK), jnp.int32),
        pltpu.VMEM((2, _CHUNK, _DIM), jnp.float32),
        pltpu.SemaphoreType.DMA,
        pltpu.SemaphoreType.DMA,
    ],
)
def _gather_kernel(table_hbm, tok_hbm, out_hbm, idx_v, rows_v, gsem, wsem):
    wid = lax.axis_index("s") * _NC + lax.axis_index("c")
    base = wid * _B_PER_W

    pltpu.sync_copy(tok_hbm.at[wid], idx_v)

    pltpu.async_copy(table_hbm.at[idx_v.at[0]], rows_v.at[0], gsem)

    def chunk_body(i, _):
        slot = lax.rem(i, 2)
        nxt = lax.rem(i + 1, 2)

        @pl.when(i >= 1)
        def _():
            pltpu.make_async_copy(
                rows_v.at[nxt],
                out_hbm.at[pl.ds(base + (i - 1) * _CHUNK, _CHUNK), pl.ds(0, _DIM)],
                wsem,
            ).wait()

        @pl.when(i + 1 < _N_CHUNKS)
        def _():
            pltpu.async_copy(
                table_hbm.at[idx_v.at[i + 1]], rows_v.at[nxt], gsem
            )

        pltpu.make_async_copy(
            table_hbm.at[idx_v.at[i]], rows_v.at[slot], gsem
        ).wait()

        pltpu.async_copy(
            rows_v.at[slot],
            out_hbm.at[pl.ds(base + i * _CHUNK, _CHUNK), pl.ds(0, _DIM)],
            wsem,
        )
        return 0

    lax.fori_loop(0, _N_CHUNKS, chunk_body, 0)

    pltpu.make_async_copy(
        rows_v.at[(_N_CHUNKS - 1) % 2],
        out_hbm.at[pl.ds(base + (_N_CHUNKS - 1) * _CHUNK, _CHUNK), pl.ds(0, _DIM)],
        wsem,
    ).wait()


def kernel(tokens, embedding_weight):
    tab_fmt = _format(embedding_weight.T)
    tab = tab_fmt.reshape(2 * _VOCAB, _DIM)
    tok = (tokens.astype(jnp.int32) * 2).reshape(_NW, _N_CHUNKS, _CHUNK)
    out = _gather_kernel(tab, tok)
    return out.reshape(_BATCH, _HIST, 2 * _DIM)[:, :, :_DIM]


# speedup vs baseline: 5.1494x; 1.0135x over previous
import functools

import jax
import jax.numpy as jnp
from jax import lax
from jax.experimental import pallas as pl
from jax.experimental.pallas import tpu as pltpu
from jax.experimental.pallas import tpu_sc as plsc

_VOCAB = 1000000
_DIM = 64
_BATCH = 4096
_HIST = 200

_NC = 2
_NS = 16
_NW = _NC * _NS

_B_TOTAL = _BATCH * _HIST
_B_PER_W = _B_TOTAL // _NW
_CHUNK = 128
_N_CHUNKS = _B_PER_W // _CHUNK

_FC = 25600
_FGRID = -(-_VOCAB // _FC)


def _fmt_body(tT_ref, out_ref):
    x = tT_ref[...]
    out_ref[:, 0:_DIM] = x.T


_format = pl.pallas_call(
    _fmt_body,
    grid=(_FGRID,),
    in_specs=[pl.BlockSpec((_DIM, _FC), lambda i: (0, i))],
    out_specs=pl.BlockSpec((_FC, 128), lambda i: (i, 0)),
    out_shape=jax.ShapeDtypeStruct((_VOCAB, 128), jnp.float32),
)


@functools.partial(
    pl.kernel,
    out_type=jax.ShapeDtypeStruct((_B_TOTAL, 2 * _DIM), jnp.float32),
    mesh=plsc.VectorSubcoreMesh(
        core_axis_name="c", subcore_axis_name="s", num_cores=_NC, num_subcores=_NS
    ),
    compiler_params=pltpu.CompilerParams(use_tc_tiling_on_sc=False),
    scratch_types=[
        pltpu.VMEM((_N_CHUNKS, _CHUNK), jnp.int32),
        pltpu.VMEM((2, _CHUNK, _DIM), jnp.float32),
        pltpu.SemaphoreType.DMA,
        pltpu.SemaphoreType.DMA,
    ],
)
def _gather_kernel(table_hbm, tok_hbm, out_hbm, idx_v, rows_v, gsem, wsem):
    wid = lax.axis_index("s") * _NC + lax.axis_index("c")
    base = wid * _B_PER_W

    pltpu.sync_copy(tok_hbm.at[wid], idx_v)

    pltpu.async_copy(table_hbm.at[idx_v.at[0]], rows_v.at[0], gsem)

    def chunk_body(i, _):
        slot = lax.rem(i, 2)
        nxt = lax.rem(i + 1, 2)

        @pl.when(i >= 1)
        def _():
            pltpu.make_async_copy(
                rows_v.at[nxt],
                out_hbm.at[pl.ds(base + (i - 1) * _CHUNK, _CHUNK), pl.ds(0, _DIM)],
                wsem,
            ).wait()

        @pl.when(i + 1 < _N_CHUNKS)
        def _():
            pltpu.async_copy(
                table_hbm.at[idx_v.at[i + 1]], rows_v.at[nxt], gsem
            )

        pltpu.make_async_copy(
            table_hbm.at[idx_v.at[i]], rows_v.at[slot], gsem
        ).wait()

        pltpu.async_copy(
            rows_v.at[slot],
            out_hbm.at[pl.ds(base + i * _CHUNK, _CHUNK), pl.ds(0, _DIM)],
            wsem,
        )
        return 0

    lax.fori_loop(0, _N_CHUNKS, chunk_body, 0)

    pltpu.make_async_copy(
        rows_v.at[(_N_CHUNKS - 1) % 2],
        out_hbm.at[pl.ds(base + (_N_CHUNKS - 1) * _CHUNK, _CHUNK), pl.ds(0, _DIM)],
        wsem,
    ).wait()


def kernel(tokens, embedding_weight):
    tab_fmt = _format(embedding_weight.T)
    tab = tab_fmt.reshape(2 * _VOCAB, _DIM)
    tok = (tokens.astype(jnp.int32) * 2).reshape(_NW, _N_CHUNKS, _CHUNK)
    out = _gather_kernel(tab, tok)
    return out.reshape(_BATCH, _HIST, 2 * _DIM)[:, :, :_DIM]


# speedup vs baseline: 5.1598x; 1.0020x over previous
import functools

import jax
import jax.numpy as jnp
from jax import lax
from jax.experimental import pallas as pl
from jax.experimental.pallas import tpu as pltpu
from jax.experimental.pallas import tpu_sc as plsc

_VOCAB = 1000000
_DIM = 64
_BATCH = 4096
_HIST = 200

_NC = 2
_NS = 16
_NW = _NC * _NS

_B_TOTAL = _BATCH * _HIST
_B_PER_W = _B_TOTAL // _NW
_CHUNK = 128
_N_CHUNKS = _B_PER_W // _CHUNK

_FC = 32000
_FGRID = -(-_VOCAB // _FC)


def _fmt_body(tT_ref, out_ref):
    x = tT_ref[...]
    out_ref[:, 0:_DIM] = x.T


_format = pl.pallas_call(
    _fmt_body,
    grid=(_FGRID,),
    in_specs=[pl.BlockSpec((_DIM, _FC), lambda i: (0, i))],
    out_specs=pl.BlockSpec((_FC, 128), lambda i: (i, 0)),
    out_shape=jax.ShapeDtypeStruct((_VOCAB, 128), jnp.float32),
)


@functools.partial(
    pl.kernel,
    out_type=jax.ShapeDtypeStruct((_B_TOTAL, 2 * _DIM), jnp.float32),
    mesh=plsc.VectorSubcoreMesh(
        core_axis_name="c", subcore_axis_name="s", num_cores=_NC, num_subcores=_NS
    ),
    compiler_params=pltpu.CompilerParams(use_tc_tiling_on_sc=False),
    scratch_types=[
        pltpu.VMEM((_N_CHUNKS, _CHUNK), jnp.int32),
        pltpu.VMEM((2, _CHUNK, _DIM), jnp.float32),
        pltpu.SemaphoreType.DMA,
        pltpu.SemaphoreType.DMA,
    ],
)
def _gather_kernel(table_hbm, tok_hbm, out_hbm, idx_v, rows_v, gsem, wsem):
    wid = lax.axis_index("s") * _NC + lax.axis_index("c")
    base = wid * _B_PER_W

    pltpu.sync_copy(tok_hbm.at[wid], idx_v)

    pltpu.async_copy(table_hbm.at[idx_v.at[0]], rows_v.at[0], gsem)

    def chunk_body(i, _):
        slot = lax.rem(i, 2)
        nxt = lax.rem(i + 1, 2)

        @pl.when(i >= 1)
        def _():
            pltpu.make_async_copy(
                rows_v.at[nxt],
                out_hbm.at[pl.ds(base + (i - 1) * _CHUNK, _CHUNK), pl.ds(0, _DIM)],
                wsem,
            ).wait()

        @pl.when(i + 1 < _N_CHUNKS)
        def _():
            pltpu.async_copy(
                table_hbm.at[idx_v.at[i + 1]], rows_v.at[nxt], gsem
            )

        pltpu.make_async_copy(
            table_hbm.at[idx_v.at[i]], rows_v.at[slot], gsem
        ).wait()

        pltpu.async_copy(
            rows_v.at[slot],
            out_hbm.at[pl.ds(base + i * _CHUNK, _CHUNK), pl.ds(0, _DIM)],
            wsem,
        )
        return 0

    lax.fori_loop(0, _N_CHUNKS, chunk_body, 0)

    pltpu.make_async_copy(
        rows_v.at[(_N_CHUNKS - 1) % 2],
        out_hbm.at[pl.ds(base + (_N_CHUNKS - 1) * _CHUNK, _CHUNK), pl.ds(0, _DIM)],
        wsem,
    ).wait()


def kernel(tokens, embedding_weight):
    tab_fmt = _format(embedding_weight.T)
    tab = tab_fmt.reshape(2 * _VOCAB, _DIM)
    tok = (tokens.astype(jnp.int32) * 2).reshape(_NW, _N_CHUNKS, _CHUNK)
    out = _gather_kernel(tab, tok)
    return out.reshape(_BATCH, _HIST, 2 * _DIM)[:, :, :_DIM]
